# baseline (speedup 1.0000x reference)
import functools

import jax
import jax.numpy as jnp
from jax import lax
from jax.experimental import pallas as pl
from jax.experimental.pallas import tpu as pltpu
from jax.experimental.pallas import tpu_sc as plsc

N = 10000
E = 320000
DF = 128
DE = 16
TD = 16
MD = 128
NC = 2
NS = 16
NW = NC * NS
EPT = E // NW
CHUNK = 40
NCHUNK = EPT // CHUNK
NSLOT = 10240


_INV2PI = 0.15915494309189535
_P2HI = 6.2831854820251465
_P2LO = -1.7484556000744883e-07
_COSC = (9.999999979337e-01, -4.999999989937e-01, 4.166666653509e-02,
         -1.388888828899e-03, 2.480156072051e-05, -2.755661835836e-07,
         2.086501806819e-09, -1.135289707997e-11, 4.128381892481e-14)


def _fast_cos(y):
    k = jnp.round(y * _INV2PI)
    r = y - k * _P2HI
    r = r - k * _P2LO
    r2 = r * r
    acc = jnp.full_like(r2, _COSC[8])
    for i in range(7, -1, -1):
        acc = acc * r2 + _COSC[i]
    return acc


def _q_body(twt_ref, ea_ref, wea_ref, wte_ref, b1_ref, o_ref):
    tet = _fast_cos(twt_ref[...])
    q = jnp.dot(ea_ref[...], wea_ref[...], preferred_element_type=jnp.float32)
    q += lax.dot_general(tet, wte_ref[...], (((0,), (0,)), ((), ())),
                         preferred_element_type=jnp.float32)
    o_ref[...] = q + b1_ref[...]


def _compute_q(twt, ea, wea_t, wte_t, b1r, e0, ne):
    BE = 3200
    g0 = e0 // BE
    grid = ne // BE
    return pl.pallas_call(
        _q_body,
        grid=(grid,),
        in_specs=[
            pl.BlockSpec((TD, BE), lambda i: (0, g0 + i)),
            pl.BlockSpec((BE, DE), lambda i: (g0 + i, 0)),
            pl.BlockSpec((DE, MD), lambda i: (0, 0)),
            pl.BlockSpec((TD, MD), lambda i: (0, 0)),
            pl.BlockSpec((1, MD), lambda i: (0, 0)),
        ],
        out_specs=pl.BlockSpec((BE, MD), lambda i: (i, 0)),
        out_shape=jax.ShapeDtypeStruct((ne, MD), jnp.float32),
    )(twt, ea, wea_t, wte_t, b1r)


def _p_body(x_ref, w_ref, o_ref):
    o_ref[...] = jnp.dot(x_ref[...], w_ref[...],
                         preferred_element_type=jnp.float32)


def _compute_p(x, w1x_t):
    BN = 2000
    return pl.pallas_call(
        _p_body,
        grid=(N // BN,),
        in_specs=[
            pl.BlockSpec((BN, DF), lambda i: (i, 0)),
            pl.BlockSpec((DF, MD), lambda i: (0, 0)),
        ],
        out_specs=pl.BlockSpec((BN, MD), lambda i: (i, 0)),
        out_shape=jax.ShapeDtypeStruct((N, MD), jnp.float32),
    )(x, w1x_t)


def _sc_edge_kernel(EPT, NCHUNK,
                    p_hbm, q_hbm, src_hbm, dst_hbm, z_hbm, out_hbm, cnt_hbm,
                    src0, src1, dst0, dst1, sdst0, sdst1,
                    rows0, rows1, q0, q1, h0, h1, cloc_v, acc_sh,
                    si0, si1, sg0, sg1, sq0, sq1, ss0, ss1):
    src_v = (src0, src1)
    dst_v = (dst0, dst1)
    sdst_v = (sdst0, sdst1)
    rows_v = (rows0, rows1)
    q_v = (q0, q1)
    h_v = (h0, h1)
    sem_i = (si0, si1)
    sem_g = (sg0, sg1)
    sem_q = (sq0, sq1)
    sem_s = (ss0, ss1)
    cid = lax.axis_index("c")
    sid = lax.axis_index("s")
    wid = sid * NC + cid
    base = wid * EPT

    NSL = N // 10

    @pl.loop(0, NSLOT // 16)
    def _zcnt(i):
        cloc_v[pl.ds(i * 16, 16)] = jnp.zeros((16,), jnp.float32)

    @pl.when(sid < 10)
    def _zcp():
        pltpu.sync_copy(z_hbm.at[pl.ds(sid * NSL, NSL)],
                        acc_sh.at[pl.ds(sid * NSL, NSL)])

    plsc.subcore_barrier()

    ones = jnp.full((16,), 1.0, jnp.float32)
    lane = lax.iota(jnp.int32, 16)
    tailmask = lane >= 8

    def _idx_copies(j, s):
        off = base + j * CHUNK
        return (pltpu.make_async_copy(src_hbm.at[pl.ds(off, CHUNK)], src_v[s],
                                      sem_i[s]),
                pltpu.make_async_copy(dst_hbm.at[pl.ds(off, CHUNK)], dst_v[s],
                                      sem_i[s]))

    def _gather_copy(s):
        return pltpu.make_async_copy(p_hbm.at[src_v[s]], rows_v[s], sem_g[s])

    def _qload_copy(j, s):
        off = base + j * CHUNK
        return pltpu.make_async_copy(q_hbm.at[pl.ds(off, CHUNK)], q_v[s],
                                     sem_q[s])

    def _chunk_body(jj, j, s, first, more1, more2):
        @pl.when(~first)
        def _w0():
            pltpu.make_async_copy(h_v[s], acc_sh.at[sdst_v[s]],
                                  sem_s[s]).wait()
        _gather_copy(s).wait()
        _qload_copy(j, s).wait()
        @pl.when(more1)
        def _l1():
            for c in _idx_copies(j + 1, 1 - s):
                c.wait()
            _gather_copy(1 - s).start()
            _qload_copy(j + 1, 1 - s).start()
        @pl.loop(0, CHUNK)
        def _row(r):
            for v in range(MD // 16):
                a = rows_v[s][r, pl.ds(v * 16, 16)]
                b = q_v[s][r, pl.ds(v * 16, 16)]
                h_v[s][r, pl.ds(v * 16, 16)] = jnp.maximum(a + b, 0.0)
        plsc.addupdate_scatter(cloc_v, [dst_v[s][pl.ds(0, 16)]], ones)
        plsc.addupdate_scatter(cloc_v, [dst_v[s][pl.ds(16, 16)]], ones)
        plsc.addupdate_scatter(cloc_v, [dst_v[s][pl.ds(24, 16)]], ones,
                               mask=tailmask)
        sdst_v[s][pl.ds(0, 16)] = dst_v[s][pl.ds(0, 16)]
        sdst_v[s][pl.ds(16, 16)] = dst_v[s][pl.ds(16, 16)]
        sdst_v[s][pl.ds(24, 16)] = dst_v[s][pl.ds(24, 16)]
        pltpu.async_copy(h_v[s], acc_sh.at[sdst_v[s]], sem_s[s], add=True)
        @pl.when(more2)
        def _l2():
            for c in _idx_copies(j + 2, s):
                c.start()

    for c in _idx_copies(0, 0):
        c.start()
    for c in _idx_copies(1, 1):
        c.start()
    for c in _idx_copies(0, 0):
        c.wait()
    _gather_copy(0).start()
    _qload_copy(0, 0).start()

    @pl.loop(0, NCHUNK // 2)
    def _step(jj):
        j0 = jj * 2
        _chunk_body(jj, j0, 0, jj == 0, j0 + 1 < NCHUNK, j0 + 2 < NCHUNK)
        _chunk_body(jj, j0 + 1, 1, jj == 0, j0 + 2 < NCHUNK, j0 + 3 < NCHUNK)

    if NCHUNK % 2:
        _chunk_body(0, NCHUNK - 1, 0, NCHUNK == 1, False, False)

    pltpu.make_async_copy(h_v[1 - NCHUNK % 2],
                          acc_sh.at[sdst_v[1 - NCHUNK % 2]],
                          sem_s[1 - NCHUNK % 2]).wait()
    pltpu.make_async_copy(h_v[NCHUNK % 2], acc_sh.at[sdst_v[NCHUNK % 2]],
                          sem_s[NCHUNK % 2]).wait()

    pltpu.sync_copy(cloc_v, cnt_hbm.at[pl.ds(wid * NSLOT, NSLOT)])
    plsc.subcore_barrier()

    @pl.when(sid < 10)
    def _dump():
        pltpu.sync_copy(acc_sh.at[pl.ds(sid * NSL, NSL)],
                        out_hbm.at[cid, pl.ds(sid * NSL, NSL)])


def _sc_edge(p, q, src, dst, z):
    ept = q.shape[0] // NW
    nchunk = ept // CHUNK
    mesh = plsc.VectorSubcoreMesh(core_axis_name="c", subcore_axis_name="s")
    kfn = pl.kernel(
        functools.partial(_sc_edge_kernel, ept, nchunk),
        mesh=mesh,
        compiler_params=pltpu.CompilerParams(use_tc_tiling_on_sc=False,
                                             needs_layout_passes=False),
        out_type=(jax.ShapeDtypeStruct((NC, N, MD), jnp.float32),
                  jax.ShapeDtypeStruct((NW * NSLOT,), jnp.float32)),
        scratch_types=[
            pltpu.VMEM((CHUNK,), jnp.int32),
            pltpu.VMEM((CHUNK,), jnp.int32),
            pltpu.VMEM((CHUNK,), jnp.int32),
            pltpu.VMEM((CHUNK,), jnp.int32),
            pltpu.VMEM((CHUNK,), jnp.int32),
            pltpu.VMEM((CHUNK,), jnp.int32),
            pltpu.VMEM((CHUNK, MD), jnp.float32),
            pltpu.VMEM((CHUNK, MD), jnp.float32),
            pltpu.VMEM((CHUNK, MD), jnp.float32),
            pltpu.VMEM((CHUNK, MD), jnp.float32),
            pltpu.VMEM((CHUNK, MD), jnp.float32),
            pltpu.VMEM((CHUNK, MD), jnp.float32),
            pltpu.VMEM((NSLOT,), jnp.float32),
            pltpu.VMEM_SHARED((N, MD), jnp.float32),
            pltpu.SemaphoreType.DMA,
            pltpu.SemaphoreType.DMA,
            pltpu.SemaphoreType.DMA,
            pltpu.SemaphoreType.DMA,
            pltpu.SemaphoreType.DMA,
            pltpu.SemaphoreType.DMA,
            pltpu.SemaphoreType.DMA,
            pltpu.SemaphoreType.DMA,
        ],
    )
    return kfn(p, q, src, dst, z)


def _csum_body(c0_ref, c1_ref, o_ref):
    s = c0_ref[0]
    for i in range(1, NW):
        s = s + c0_ref[i]
    for i in range(NW):
        s = s + c1_ref[i]
    o_ref[...] = s


def _count_reduce(cnt0, cnt1):
    return pl.pallas_call(
        _csum_body,
        grid=(1,),
        in_specs=[pl.BlockSpec((NW, NSLOT // 128, 128), lambda i: (0, 0, 0)),
                  pl.BlockSpec((NW, NSLOT // 128, 128), lambda i: (0, 0, 0))],
        out_specs=pl.BlockSpec((NSLOT // 128, 128), lambda i: (0, 0)),
        out_shape=jax.ShapeDtypeStruct((NSLOT // 128, 128), jnp.float32),
    )(cnt0, cnt1)


def _node_body(sv_ref, sw_ref, cw_ref, w2_ref, b2_ref, wih_ref, bih_ref,
               bhh_ref, wout_ref, bout_ref, o_ref):
    s = (sv_ref[0] + sv_ref[1]) + (sw_ref[0] + sw_ref[1])
    cnt = cw_ref[...]
    has = (cnt > 0.0).astype(jnp.float32)
    agg = jnp.dot(s / jnp.maximum(cnt, 1.0), w2_ref[...],
                  preferred_element_type=jnp.float32)
    agg += b2_ref[...] * has
    gi = jnp.dot(agg, wih_ref[...], preferred_element_type=jnp.float32)
    gi += bih_ref[...]
    bhh = bhh_ref[...]
    r = jax.nn.sigmoid(gi[:, 0:MD] + bhh[:, 0:MD])
    z = jax.nn.sigmoid(gi[:, MD:2 * MD] + bhh[:, MD:2 * MD])
    n = jnp.tanh(gi[:, 2 * MD:3 * MD] + r * bhh[:, 2 * MD:3 * MD])
    mem = (1.0 - z) * n
    o_ref[...] = jnp.dot(mem, wout_ref[...],
                         preferred_element_type=jnp.float32) + bout_ref[...]


def _node_final(sv, sw, cw, w2_t, b2r, wih_t, bihr, bhhr, wout_t, boutr):
    BN = 2000
    return pl.pallas_call(
        _node_body,
        grid=(N // BN,),
        in_specs=[
            pl.BlockSpec((NC, BN, MD), lambda i: (0, i, 0)),
            pl.BlockSpec((NC, BN, MD), lambda i: (0, i, 0)),
            pl.BlockSpec((BN, 1), lambda i: (i, 0)),
            pl.BlockSpec((MD, MD), lambda i: (0, 0)),
            pl.BlockSpec((1, MD), lambda i: (0, 0)),
            pl.BlockSpec((MD, 3 * MD), lambda i: (0, 0)),
            pl.BlockSpec((1, 3 * MD), lambda i: (0, 0)),
            pl.BlockSpec((1, 3 * MD), lambda i: (0, 0)),
            pl.BlockSpec((MD, DF), lambda i: (0, 0)),
            pl.BlockSpec((1, DF), lambda i: (0, 0)),
        ],
        out_specs=pl.BlockSpec((BN, DF), lambda i: (i, 0)),
        out_shape=jax.ShapeDtypeStruct((N, DF), jnp.float32),
    )(sv, sw, cw, w2_t, b2r, wih_t, bihr, bhhr, wout_t, boutr)


def kernel(x, edge_index, edge_attr, t, w_time, b_time, W1, b1, W2, b2,
           W_ih, b_ih, W_hh, b_hh, W_out, b_out):
    src = edge_index[0]
    dst = edge_index[1]
    w1x_t = W1[:, :DF].T
    wea_t = W1[:, DF:DF + DE].T
    wte_t = W1[:, DF + DE:].T
    b1r = b1.reshape(1, MD)

    twt = w_time.reshape(TD, 1) * t[None, :] + b_time.reshape(TD, 1)
    E2 = E // 2
    p = _compute_p(x, w1x_t)
    q0 = _compute_q(twt, edge_attr, wea_t, wte_t, b1r, 0, E2)
    q1 = _compute_q(twt, edge_attr, wea_t, wte_t, b1r, E2, E2)
    z = jnp.zeros((N, MD), jnp.float32)
    acc0, cnt0 = _sc_edge(p, q0, src[:E2], dst[:E2], z)
    acc1, cnt1 = _sc_edge(p, q1, src[E2:], dst[E2:], z)

    csum = _count_reduce(cnt0.reshape(NW, NSLOT // 128, 128),
                         cnt1.reshape(NW, NSLOT // 128, 128))
    cw = csum.reshape(NSLOT)[:N].reshape(N, 1)
    return _node_final(acc0, acc1, cw, W2.T, b2.reshape(1, MD), W_ih.T,
                       b_ih.reshape(1, 3 * MD), b_hh.reshape(1, 3 * MD),
                       W_out.T, b_out.reshape(1, DF))


# speedup vs baseline: 1.0379x; 1.0379x over previous
import functools

import jax
import jax.numpy as jnp
from jax import lax
from jax.experimental import pallas as pl
from jax.experimental.pallas import tpu as pltpu
from jax.experimental.pallas import tpu_sc as plsc

N = 10000
E = 320000
DF = 128
DE = 16
TD = 16
MD = 128
NC = 2
NS = 16
NW = NC * NS
EPT = E // NW
CHUNK = 40
NCHUNK = EPT // CHUNK
NSLOT = 10240


_INV2PI = 0.15915494309189535
_P2HI = 6.2831854820251465
_P2LO = -1.7484556000744883e-07
_COSC = (9.999999979337e-01, -4.999999989937e-01, 4.166666653509e-02,
         -1.388888828899e-03, 2.480156072051e-05, -2.755661835836e-07,
         2.086501806819e-09, -1.135289707997e-11, 4.128381892481e-14)


def _fast_cos(y):
    k = jnp.round(y * _INV2PI)
    r = y - k * _P2HI
    r = r - k * _P2LO
    r2 = r * r
    acc = jnp.full_like(r2, _COSC[8])
    for i in range(7, -1, -1):
        acc = acc * r2 + _COSC[i]
    return acc


def _q_body(twt_ref, ea_ref, wea_ref, wte_ref, b1_ref, o_ref):
    tet = _fast_cos(twt_ref[...])
    q = jnp.dot(ea_ref[...], wea_ref[...], preferred_element_type=jnp.float32)
    q += lax.dot_general(tet, wte_ref[...], (((0,), (0,)), ((), ())),
                         preferred_element_type=jnp.float32)
    o_ref[...] = q + b1_ref[...]


def _compute_q(twt, ea, wea_t, wte_t, b1r, e0, ne):
    BE = 3200
    g0 = e0 // BE
    grid = ne // BE
    return pl.pallas_call(
        _q_body,
        grid=(grid,),
        in_specs=[
            pl.BlockSpec((TD, BE), lambda i: (0, g0 + i)),
            pl.BlockSpec((BE, DE), lambda i: (g0 + i, 0)),
            pl.BlockSpec((DE, MD), lambda i: (0, 0)),
            pl.BlockSpec((TD, MD), lambda i: (0, 0)),
            pl.BlockSpec((1, MD), lambda i: (0, 0)),
        ],
        out_specs=pl.BlockSpec((BE, MD), lambda i: (i, 0)),
        out_shape=jax.ShapeDtypeStruct((ne, MD), jnp.float32),
    )(twt, ea, wea_t, wte_t, b1r)


def _p_body(x_ref, w_ref, o_ref):
    o_ref[...] = jnp.dot(x_ref[...], w_ref[...],
                         preferred_element_type=jnp.float32)


def _compute_p(x, w1x_t):
    BN = 2000
    return pl.pallas_call(
        _p_body,
        grid=(N // BN,),
        in_specs=[
            pl.BlockSpec((BN, DF), lambda i: (i, 0)),
            pl.BlockSpec((DF, MD), lambda i: (0, 0)),
        ],
        out_specs=pl.BlockSpec((BN, MD), lambda i: (i, 0)),
        out_shape=jax.ShapeDtypeStruct((N, MD), jnp.float32),
    )(x, w1x_t)


def _sc_edge_kernel(EPT, NCHUNK,
                    p_hbm, q_hbm, src_hbm, dst_hbm, z_hbm, out_hbm, cnt_hbm,
                    src0, src1, dst0, dst1, sdst0, sdst1,
                    rows0, rows1, q0, q1, h0, h1, cloc_v, acc_sh,
                    si0, si1, sg0, sg1, sq0, sq1, ss0, ss1):
    src_v = (src0, src1)
    dst_v = (dst0, dst1)
    sdst_v = (sdst0, sdst1)
    rows_v = (rows0, rows1)
    q_v = (q0, q1)
    h_v = (h0, h1)
    sem_i = (si0, si1)
    sem_g = (sg0, sg1)
    sem_q = (sq0, sq1)
    sem_s = (ss0, ss1)
    cid = lax.axis_index("c")
    sid = lax.axis_index("s")
    wid = sid * NC + cid
    base = wid * EPT

    NSL = N // 10

    @pl.loop(0, NSLOT // 16)
    def _zcnt(i):
        cloc_v[pl.ds(i * 16, 16)] = jnp.zeros((16,), jnp.float32)

    @pl.when(sid < 10)
    def _zcp():
        pltpu.sync_copy(z_hbm.at[pl.ds(sid * NSL, NSL)],
                        acc_sh.at[pl.ds(sid * NSL, NSL)])

    plsc.subcore_barrier()

    ones = jnp.full((16,), 1.0, jnp.float32)
    lane = lax.iota(jnp.int32, 16)
    tailmask = lane >= 8

    def _idx_copies(j, s):
        off = base + j * CHUNK
        return (pltpu.make_async_copy(src_hbm.at[pl.ds(off, CHUNK)], src_v[s],
                                      sem_i[s]),
                pltpu.make_async_copy(dst_hbm.at[pl.ds(off, CHUNK)], dst_v[s],
                                      sem_i[s]))

    def _gather_copy(s):
        return pltpu.make_async_copy(p_hbm.at[src_v[s]], rows_v[s], sem_g[s])

    def _qload_copy(j, s):
        off = base + j * CHUNK
        return pltpu.make_async_copy(q_hbm.at[pl.ds(off, CHUNK)], q_v[s],
                                     sem_q[s])

    def _chunk_body(jj, j, s, first, more1, more2):
        @pl.when(~first)
        def _w0():
            pltpu.make_async_copy(h_v[s], acc_sh.at[sdst_v[s]],
                                  sem_s[s]).wait()
        _gather_copy(s).wait()
        _qload_copy(j, s).wait()
        @pl.when(more1)
        def _l1():
            for c in _idx_copies(j + 1, 1 - s):
                c.wait()
            _gather_copy(1 - s).start()
            _qload_copy(j + 1, 1 - s).start()
        @pl.loop(0, CHUNK)
        def _row(r):
            for v in range(MD // 16):
                a = rows_v[s][r, pl.ds(v * 16, 16)]
                b = q_v[s][r, pl.ds(v * 16, 16)]
                h_v[s][r, pl.ds(v * 16, 16)] = jnp.maximum(a + b, 0.0)
        plsc.addupdate_scatter(cloc_v, [dst_v[s][pl.ds(0, 16)]], ones)
        plsc.addupdate_scatter(cloc_v, [dst_v[s][pl.ds(16, 16)]], ones)
        plsc.addupdate_scatter(cloc_v, [dst_v[s][pl.ds(24, 16)]], ones,
                               mask=tailmask)
        sdst_v[s][pl.ds(0, 16)] = dst_v[s][pl.ds(0, 16)]
        sdst_v[s][pl.ds(16, 16)] = dst_v[s][pl.ds(16, 16)]
        sdst_v[s][pl.ds(24, 16)] = dst_v[s][pl.ds(24, 16)]
        pltpu.async_copy(h_v[s], acc_sh.at[sdst_v[s]], sem_s[s], add=True)
        @pl.when(more2)
        def _l2():
            for c in _idx_copies(j + 2, s):
                c.start()

    for c in _idx_copies(0, 0):
        c.start()
    for c in _idx_copies(1, 1):
        c.start()
    for c in _idx_copies(0, 0):
        c.wait()
    _gather_copy(0).start()
    _qload_copy(0, 0).start()

    @pl.loop(0, NCHUNK // 2)
    def _step(jj):
        j0 = jj * 2
        _chunk_body(jj, j0, 0, jj == 0, j0 + 1 < NCHUNK, j0 + 2 < NCHUNK)
        _chunk_body(jj, j0 + 1, 1, jj == 0, j0 + 2 < NCHUNK, j0 + 3 < NCHUNK)

    if NCHUNK % 2:
        _chunk_body(0, NCHUNK - 1, 0, NCHUNK == 1, False, False)

    pltpu.make_async_copy(h_v[1 - NCHUNK % 2],
                          acc_sh.at[sdst_v[1 - NCHUNK % 2]],
                          sem_s[1 - NCHUNK % 2]).wait()
    pltpu.make_async_copy(h_v[NCHUNK % 2], acc_sh.at[sdst_v[NCHUNK % 2]],
                          sem_s[NCHUNK % 2]).wait()

    pltpu.sync_copy(cloc_v, cnt_hbm.at[pl.ds(wid * NSLOT, NSLOT)])
    plsc.subcore_barrier()

    @pl.when(sid < 10)
    def _dump():
        pltpu.sync_copy(acc_sh.at[pl.ds(sid * NSL, NSL)],
                        out_hbm.at[cid, pl.ds(sid * NSL, NSL)])


def _sc_edge(p, q, src, dst, z):
    ept = q.shape[0] // NW
    nchunk = ept // CHUNK
    mesh = plsc.VectorSubcoreMesh(core_axis_name="c", subcore_axis_name="s")
    kfn = pl.kernel(
        functools.partial(_sc_edge_kernel, ept, nchunk),
        mesh=mesh,
        compiler_params=pltpu.CompilerParams(use_tc_tiling_on_sc=False,
                                             needs_layout_passes=False),
        out_type=(jax.ShapeDtypeStruct((NC, N, MD), jnp.float32),
                  jax.ShapeDtypeStruct((NW * NSLOT,), jnp.float32)),
        scratch_types=[
            pltpu.VMEM((CHUNK,), jnp.int32),
            pltpu.VMEM((CHUNK,), jnp.int32),
            pltpu.VMEM((CHUNK,), jnp.int32),
            pltpu.VMEM((CHUNK,), jnp.int32),
            pltpu.VMEM((CHUNK,), jnp.int32),
            pltpu.VMEM((CHUNK,), jnp.int32),
            pltpu.VMEM((CHUNK, MD), jnp.float32),
            pltpu.VMEM((CHUNK, MD), jnp.float32),
            pltpu.VMEM((CHUNK, MD), jnp.float32),
            pltpu.VMEM((CHUNK, MD), jnp.float32),
            pltpu.VMEM((CHUNK, MD), jnp.float32),
            pltpu.VMEM((CHUNK, MD), jnp.float32),
            pltpu.VMEM((NSLOT,), jnp.float32),
            pltpu.VMEM_SHARED((N, MD), jnp.float32),
            pltpu.SemaphoreType.DMA,
            pltpu.SemaphoreType.DMA,
            pltpu.SemaphoreType.DMA,
            pltpu.SemaphoreType.DMA,
            pltpu.SemaphoreType.DMA,
            pltpu.SemaphoreType.DMA,
            pltpu.SemaphoreType.DMA,
            pltpu.SemaphoreType.DMA,
        ],
    )
    return kfn(p, q, src, dst, z)


def _csum_body(c0_ref, c1_ref, o_ref):
    s = c0_ref[0]
    for i in range(1, NW):
        s = s + c0_ref[i]
    for i in range(NW):
        s = s + c1_ref[i]
    o_ref[...] = s


def _count_reduce(cnt0, cnt1):
    return pl.pallas_call(
        _csum_body,
        grid=(1,),
        in_specs=[pl.BlockSpec((NW, NSLOT // 128, 128), lambda i: (0, 0, 0)),
                  pl.BlockSpec((NW, NSLOT // 128, 128), lambda i: (0, 0, 0))],
        out_specs=pl.BlockSpec((NSLOT // 128, 128), lambda i: (0, 0)),
        out_shape=jax.ShapeDtypeStruct((NSLOT // 128, 128), jnp.float32),
    )(cnt0, cnt1)


def _node_body(sv_ref, sw_ref, cw_ref, w2_ref, b2_ref, wih_ref, bih_ref,
               bhh_ref, wout_ref, bout_ref, o_ref):
    s = (sv_ref[0] + sv_ref[1]) + (sw_ref[0] + sw_ref[1])
    cnt = cw_ref[...]
    has = (cnt > 0.0).astype(jnp.float32)
    agg = jnp.dot(s / jnp.maximum(cnt, 1.0), w2_ref[...],
                  preferred_element_type=jnp.float32)
    agg += b2_ref[...] * has
    gi = jnp.dot(agg, wih_ref[...], preferred_element_type=jnp.float32)
    gi += bih_ref[...]
    bhh = bhh_ref[...]
    r = jax.nn.sigmoid(gi[:, 0:MD] + bhh[:, 0:MD])
    z = jax.nn.sigmoid(gi[:, MD:2 * MD] + bhh[:, MD:2 * MD])
    n = jnp.tanh(gi[:, 2 * MD:3 * MD] + r * bhh[:, 2 * MD:3 * MD])
    mem = (1.0 - z) * n
    o_ref[...] = jnp.dot(mem, wout_ref[...],
                         preferred_element_type=jnp.float32) + bout_ref[...]


def _node_final(sv, sw, cw, w2_t, b2r, wih_t, bihr, bhhr, wout_t, boutr):
    BN = 2000
    return pl.pallas_call(
        _node_body,
        grid=(N // BN,),
        in_specs=[
            pl.BlockSpec((NC, BN, MD), lambda i: (0, i, 0)),
            pl.BlockSpec((NC, BN, MD), lambda i: (0, i, 0)),
            pl.BlockSpec((BN, 1), lambda i: (i, 0)),
            pl.BlockSpec((MD, MD), lambda i: (0, 0)),
            pl.BlockSpec((1, MD), lambda i: (0, 0)),
            pl.BlockSpec((MD, 3 * MD), lambda i: (0, 0)),
            pl.BlockSpec((1, 3 * MD), lambda i: (0, 0)),
            pl.BlockSpec((1, 3 * MD), lambda i: (0, 0)),
            pl.BlockSpec((MD, DF), lambda i: (0, 0)),
            pl.BlockSpec((1, DF), lambda i: (0, 0)),
        ],
        out_specs=pl.BlockSpec((BN, DF), lambda i: (i, 0)),
        out_shape=jax.ShapeDtypeStruct((N, DF), jnp.float32),
    )(sv, sw, cw, w2_t, b2r, wih_t, bihr, bhhr, wout_t, boutr)


def kernel(x, edge_index, edge_attr, t, w_time, b_time, W1, b1, W2, b2,
           W_ih, b_ih, W_hh, b_hh, W_out, b_out):
    src = edge_index[0]
    dst = edge_index[1]
    w1x_t = W1[:, :DF].T
    wea_t = W1[:, DF:DF + DE].T
    wte_t = W1[:, DF + DE:].T
    b1r = b1.reshape(1, MD)

    twt = w_time.reshape(TD, 1) * t[None, :] + b_time.reshape(TD, 1)
    E2 = 128000
    p = _compute_p(x, w1x_t)
    q0 = _compute_q(twt, edge_attr, wea_t, wte_t, b1r, 0, E2)
    q1 = _compute_q(twt, edge_attr, wea_t, wte_t, b1r, E2, E - E2)
    z = jnp.zeros((N, MD), jnp.float32)
    acc0, cnt0 = _sc_edge(p, q0, src[:E2], dst[:E2], z)
    acc1, cnt1 = _sc_edge(p, q1, src[E2:], dst[E2:], z)

    csum = _count_reduce(cnt0.reshape(NW, NSLOT // 128, 128),
                         cnt1.reshape(NW, NSLOT // 128, 128))
    cw = csum.reshape(NSLOT)[:N].reshape(N, 1)
    return _node_final(acc0, acc1, cw, W2.T, b2.reshape(1, MD), W_ih.T,
                       b_ih.reshape(1, 3 * MD), b_hh.reshape(1, 3 * MD),
                       W_out.T, b_out.reshape(1, DF))


# speedup vs baseline: 1.0435x; 1.0054x over previous
import functools

import jax
import jax.numpy as jnp
from jax import lax
from jax.experimental import pallas as pl
from jax.experimental.pallas import tpu as pltpu
from jax.experimental.pallas import tpu_sc as plsc

N = 10000
E = 320000
DF = 128
DE = 16
TD = 16
MD = 128
NC = 2
NS = 16
NW = NC * NS
EPT = E // NW
CHUNK = 40
NCHUNK = EPT // CHUNK
NSLOT = 10240


_INV2PI = 0.15915494309189535
_P2HI = 6.2831854820251465
_P2LO = -1.7484556000744883e-07
_COSC = (9.999999979337e-01, -4.999999989937e-01, 4.166666653509e-02,
         -1.388888828899e-03, 2.480156072051e-05, -2.755661835836e-07,
         2.086501806819e-09, -1.135289707997e-11, 4.128381892481e-14)


def _fast_cos(y):
    k = jnp.round(y * _INV2PI)
    r = y - k * _P2HI
    r = r - k * _P2LO
    r2 = r * r
    acc = jnp.full_like(r2, _COSC[8])
    for i in range(7, -1, -1):
        acc = acc * r2 + _COSC[i]
    return acc


def _q_body(twt_ref, ea_ref, wea_ref, wte_ref, b1_ref, o_ref):
    tet = _fast_cos(twt_ref[...])
    q = jnp.dot(ea_ref[...], wea_ref[...], preferred_element_type=jnp.float32)
    q += lax.dot_general(tet, wte_ref[...], (((0,), (0,)), ((), ())),
                         preferred_element_type=jnp.float32)
    o_ref[...] = q + b1_ref[...]


def _compute_q(twt, ea, wea_t, wte_t, b1r, e0, ne):
    BE = 3200
    g0 = e0 // BE
    grid = ne // BE
    return pl.pallas_call(
        _q_body,
        grid=(grid,),
        in_specs=[
            pl.BlockSpec((TD, BE), lambda i: (0, g0 + i)),
            pl.BlockSpec((BE, DE), lambda i: (g0 + i, 0)),
            pl.BlockSpec((DE, MD), lambda i: (0, 0)),
            pl.BlockSpec((TD, MD), lambda i: (0, 0)),
            pl.BlockSpec((1, MD), lambda i: (0, 0)),
        ],
        out_specs=pl.BlockSpec((BE, MD), lambda i: (i, 0)),
        out_shape=jax.ShapeDtypeStruct((ne, MD), jnp.float32),
    )(twt, ea, wea_t, wte_t, b1r)


def _p_body(x_ref, w_ref, o_ref):
    o_ref[...] = jnp.dot(x_ref[...], w_ref[...],
                         preferred_element_type=jnp.float32)


def _compute_p(x, w1x_t):
    BN = 2000
    return pl.pallas_call(
        _p_body,
        grid=(N // BN,),
        in_specs=[
            pl.BlockSpec((BN, DF), lambda i: (i, 0)),
            pl.BlockSpec((DF, MD), lambda i: (0, 0)),
        ],
        out_specs=pl.BlockSpec((BN, MD), lambda i: (i, 0)),
        out_shape=jax.ShapeDtypeStruct((N, MD), jnp.float32),
    )(x, w1x_t)


def _sc_edge_kernel(EPT, NCHUNK,
                    p_hbm, q_hbm, src_hbm, dst_hbm, z_hbm, out_hbm, cnt_hbm,
                    src0, src1, dst0, dst1, sdst0, sdst1,
                    rows0, rows1, q0, q1, h0, h1, cloc_v, acc_sh,
                    si0, si1, sg0, sg1, sq0, sq1, ss0, ss1):
    src_v = (src0, src1)
    dst_v = (dst0, dst1)
    sdst_v = (sdst0, sdst1)
    rows_v = (rows0, rows1)
    q_v = (q0, q1)
    h_v = (h0, h1)
    sem_i = (si0, si1)
    sem_g = (sg0, sg1)
    sem_q = (sq0, sq1)
    sem_s = (ss0, ss1)
    cid = lax.axis_index("c")
    sid = lax.axis_index("s")
    wid = sid * NC + cid
    base = wid * EPT

    NSL = N // 10

    @pl.loop(0, NSLOT // 16)
    def _zcnt(i):
        cloc_v[pl.ds(i * 16, 16)] = jnp.zeros((16,), jnp.float32)

    @pl.when(sid < 10)
    def _zcp():
        pltpu.sync_copy(z_hbm.at[pl.ds(sid * NSL, NSL)],
                        acc_sh.at[pl.ds(sid * NSL, NSL)])

    plsc.subcore_barrier()

    ones = jnp.full((16,), 1.0, jnp.float32)
    lane = lax.iota(jnp.int32, 16)
    tailmask = lane >= 8

    def _idx_copies(j, s):
        off = base + j * CHUNK
        return (pltpu.make_async_copy(src_hbm.at[pl.ds(off, CHUNK)], src_v[s],
                                      sem_i[s]),
                pltpu.make_async_copy(dst_hbm.at[pl.ds(off, CHUNK)], dst_v[s],
                                      sem_i[s]))

    def _gather_copy(s):
        return pltpu.make_async_copy(p_hbm.at[src_v[s]], rows_v[s], sem_g[s])

    def _qload_copy(j, s):
        off = base + j * CHUNK
        return pltpu.make_async_copy(q_hbm.at[pl.ds(off, CHUNK)], q_v[s],
                                     sem_q[s])

    def _chunk_body(jj, j, s, first, more1, more2):
        @pl.when(~first)
        def _w0():
            pltpu.make_async_copy(h_v[s], acc_sh.at[sdst_v[s]],
                                  sem_s[s]).wait()
        _gather_copy(s).wait()
        _qload_copy(j, s).wait()
        @pl.when(more1)
        def _l1():
            for c in _idx_copies(j + 1, 1 - s):
                c.wait()
            _gather_copy(1 - s).start()
            _qload_copy(j + 1, 1 - s).start()
        @pl.loop(0, CHUNK)
        def _row(r):
            for v in range(MD // 16):
                a = rows_v[s][r, pl.ds(v * 16, 16)]
                b = q_v[s][r, pl.ds(v * 16, 16)]
                h_v[s][r, pl.ds(v * 16, 16)] = jnp.maximum(a + b, 0.0)
        plsc.addupdate_scatter(cloc_v, [dst_v[s][pl.ds(0, 16)]], ones)
        plsc.addupdate_scatter(cloc_v, [dst_v[s][pl.ds(16, 16)]], ones)
        plsc.addupdate_scatter(cloc_v, [dst_v[s][pl.ds(24, 16)]], ones,
                               mask=tailmask)
        sdst_v[s][pl.ds(0, 16)] = dst_v[s][pl.ds(0, 16)]
        sdst_v[s][pl.ds(16, 16)] = dst_v[s][pl.ds(16, 16)]
        sdst_v[s][pl.ds(24, 16)] = dst_v[s][pl.ds(24, 16)]
        pltpu.async_copy(h_v[s], acc_sh.at[sdst_v[s]], sem_s[s], add=True)
        @pl.when(more2)
        def _l2():
            for c in _idx_copies(j + 2, s):
                c.start()

    for c in _idx_copies(0, 0):
        c.start()
    for c in _idx_copies(1, 1):
        c.start()
    for c in _idx_copies(0, 0):
        c.wait()
    _gather_copy(0).start()
    _qload_copy(0, 0).start()

    @pl.loop(0, NCHUNK // 2)
    def _step(jj):
        j0 = jj * 2
        _chunk_body(jj, j0, 0, jj == 0, j0 + 1 < NCHUNK, j0 + 2 < NCHUNK)
        _chunk_body(jj, j0 + 1, 1, jj == 0, j0 + 2 < NCHUNK, j0 + 3 < NCHUNK)

    if NCHUNK % 2:
        _chunk_body(0, NCHUNK - 1, 0, NCHUNK == 1, False, False)

    pltpu.make_async_copy(h_v[1 - NCHUNK % 2],
                          acc_sh.at[sdst_v[1 - NCHUNK % 2]],
                          sem_s[1 - NCHUNK % 2]).wait()
    pltpu.make_async_copy(h_v[NCHUNK % 2], acc_sh.at[sdst_v[NCHUNK % 2]],
                          sem_s[NCHUNK % 2]).wait()

    pltpu.sync_copy(cloc_v, cnt_hbm.at[pl.ds(wid * NSLOT, NSLOT)])
    plsc.subcore_barrier()

    @pl.when(sid < 10)
    def _dump():
        pltpu.sync_copy(acc_sh.at[pl.ds(sid * NSL, NSL)],
                        out_hbm.at[cid, pl.ds(sid * NSL, NSL)])


def _sc_edge(p, q, src, dst, z):
    ept = q.shape[0] // NW
    nchunk = ept // CHUNK
    mesh = plsc.VectorSubcoreMesh(core_axis_name="c", subcore_axis_name="s")
    kfn = pl.kernel(
        functools.partial(_sc_edge_kernel, ept, nchunk),
        mesh=mesh,
        compiler_params=pltpu.CompilerParams(use_tc_tiling_on_sc=False,
                                             needs_layout_passes=False),
        out_type=(jax.ShapeDtypeStruct((NC, N, MD), jnp.float32),
                  jax.ShapeDtypeStruct((NW * NSLOT,), jnp.float32)),
        scratch_types=[
            pltpu.VMEM((CHUNK,), jnp.int32),
            pltpu.VMEM((CHUNK,), jnp.int32),
            pltpu.VMEM((CHUNK,), jnp.int32),
            pltpu.VMEM((CHUNK,), jnp.int32),
            pltpu.VMEM((CHUNK,), jnp.int32),
            pltpu.VMEM((CHUNK,), jnp.int32),
            pltpu.VMEM((CHUNK, MD), jnp.float32),
            pltpu.VMEM((CHUNK, MD), jnp.float32),
            pltpu.VMEM((CHUNK, MD), jnp.float32),
            pltpu.VMEM((CHUNK, MD), jnp.float32),
            pltpu.VMEM((CHUNK, MD), jnp.float32),
            pltpu.VMEM((CHUNK, MD), jnp.float32),
            pltpu.VMEM((NSLOT,), jnp.float32),
            pltpu.VMEM_SHARED((N, MD), jnp.float32),
            pltpu.SemaphoreType.DMA,
            pltpu.SemaphoreType.DMA,
            pltpu.SemaphoreType.DMA,
            pltpu.SemaphoreType.DMA,
            pltpu.SemaphoreType.DMA,
            pltpu.SemaphoreType.DMA,
            pltpu.SemaphoreType.DMA,
            pltpu.SemaphoreType.DMA,
        ],
    )
    return kfn(p, q, src, dst, z)


def _csum_body(c0_ref, c1_ref, o_ref):
    s = c0_ref[0]
    for i in range(1, NW):
        s = s + c0_ref[i]
    for i in range(NW):
        s = s + c1_ref[i]
    o_ref[...] = s


def _count_reduce(cnt0, cnt1):
    return pl.pallas_call(
        _csum_body,
        grid=(1,),
        in_specs=[pl.BlockSpec((NW, NSLOT // 128, 128), lambda i: (0, 0, 0)),
                  pl.BlockSpec((NW, NSLOT // 128, 128), lambda i: (0, 0, 0))],
        out_specs=pl.BlockSpec((NSLOT // 128, 128), lambda i: (0, 0)),
        out_shape=jax.ShapeDtypeStruct((NSLOT // 128, 128), jnp.float32),
    )(cnt0, cnt1)


def _node_body(sv_ref, sw_ref, cw_ref, w2_ref, b2_ref, wih_ref, bih_ref,
               bhh_ref, wout_ref, bout_ref, o_ref):
    s = (sv_ref[0] + sv_ref[1]) + (sw_ref[0] + sw_ref[1])
    cnt = cw_ref[...]
    has = (cnt > 0.0).astype(jnp.float32)
    agg = jnp.dot(s / jnp.maximum(cnt, 1.0), w2_ref[...],
                  preferred_element_type=jnp.float32)
    agg += b2_ref[...] * has
    gi = jnp.dot(agg, wih_ref[...], preferred_element_type=jnp.float32)
    gi += bih_ref[...]
    bhh = bhh_ref[...]
    r = jax.nn.sigmoid(gi[:, 0:MD] + bhh[:, 0:MD])
    z = jax.nn.sigmoid(gi[:, MD:2 * MD] + bhh[:, MD:2 * MD])
    n = jnp.tanh(gi[:, 2 * MD:3 * MD] + r * bhh[:, 2 * MD:3 * MD])
    mem = (1.0 - z) * n
    o_ref[...] = jnp.dot(mem, wout_ref[...],
                         preferred_element_type=jnp.float32) + bout_ref[...]


def _node_final(sv, sw, cw, w2_t, b2r, wih_t, bihr, bhhr, wout_t, boutr):
    BN = 2000
    return pl.pallas_call(
        _node_body,
        grid=(N // BN,),
        in_specs=[
            pl.BlockSpec((NC, BN, MD), lambda i: (0, i, 0)),
            pl.BlockSpec((NC, BN, MD), lambda i: (0, i, 0)),
            pl.BlockSpec((BN, 1), lambda i: (i, 0)),
            pl.BlockSpec((MD, MD), lambda i: (0, 0)),
            pl.BlockSpec((1, MD), lambda i: (0, 0)),
            pl.BlockSpec((MD, 3 * MD), lambda i: (0, 0)),
            pl.BlockSpec((1, 3 * MD), lambda i: (0, 0)),
            pl.BlockSpec((1, 3 * MD), lambda i: (0, 0)),
            pl.BlockSpec((MD, DF), lambda i: (0, 0)),
            pl.BlockSpec((1, DF), lambda i: (0, 0)),
        ],
        out_specs=pl.BlockSpec((BN, DF), lambda i: (i, 0)),
        out_shape=jax.ShapeDtypeStruct((N, DF), jnp.float32),
    )(sv, sw, cw, w2_t, b2r, wih_t, bihr, bhhr, wout_t, boutr)


def kernel(x, edge_index, edge_attr, t, w_time, b_time, W1, b1, W2, b2,
           W_ih, b_ih, W_hh, b_hh, W_out, b_out):
    src = edge_index[0]
    dst = edge_index[1]
    w1x_t = W1[:, :DF].T
    wea_t = W1[:, DF:DF + DE].T
    wte_t = W1[:, DF + DE:].T
    b1r = b1.reshape(1, MD)

    twt = w_time.reshape(TD, 1) * t[None, :] + b_time.reshape(TD, 1)
    E2 = 115200
    p = _compute_p(x, w1x_t)
    q0 = _compute_q(twt, edge_attr, wea_t, wte_t, b1r, 0, E2)
    q1 = _compute_q(twt, edge_attr, wea_t, wte_t, b1r, E2, E - E2)
    z = jnp.zeros((N, MD), jnp.float32)
    acc0, cnt0 = _sc_edge(p, q0, src[:E2], dst[:E2], z)
    acc1, cnt1 = _sc_edge(p, q1, src[E2:], dst[E2:], z)

    csum = _count_reduce(cnt0.reshape(NW, NSLOT // 128, 128),
                         cnt1.reshape(NW, NSLOT // 128, 128))
    cw = csum.reshape(NSLOT)[:N].reshape(N, 1)
    return _node_final(acc0, acc1, cw, W2.T, b2.reshape(1, MD), W_ih.T,
                       b_ih.reshape(1, 3 * MD), b_hh.reshape(1, 3 * MD),
                       W_out.T, b_out.reshape(1, DF))
